# baseline (speedup 1.0000x reference)
import functools

import jax
import jax.numpy as jnp
from jax.experimental import pallas as pl
from jax.experimental.pallas import tpu as pltpu

CONTEXT = 96


def _dist_kernel(k_ref, c_ref, out_ref):
    k = k_ref[...]
    c = c_ref[...]
    ksq = jnp.sum(k * k, axis=-1, keepdims=True)
    csq = jnp.sum(c * c, axis=-1)
    cross = jax.lax.dot_general(
        k, c, (((1,), (1,)), ((), ())),
        preferred_element_type=jnp.float32)
    out_ref[...] = ksq - 2.0 * cross + csq[None, :]


def _silu(x):
    return x * jax.nn.sigmoid(x)


def _layer_norm(x, g, b, eps=1e-5):
    mu = jnp.mean(x, axis=-1, keepdims=True)
    var = jnp.var(x, axis=-1, keepdims=True)
    return (x - mu) / jnp.sqrt(var + eps) * g + b


def kernel(k, candidate_embeddings, candidate_y, label_emb, kp_ln_g, kp_ln_b,
           kp_w1, kp_b1, kp_w2, kp_b2, pr_w1, pr_b1, pr_w2, pr_b2):
    B, d = k.shape
    N = candidate_embeddings.shape[0]
    NT = 2048
    N_pad = ((N + NT - 1) // NT) * NT
    cand_pad = jnp.pad(candidate_embeddings, ((0, N_pad - N), (0, 0)),
                       constant_values=1e3)
    BT = 256
    dists = pl.pallas_call(
        _dist_kernel,
        grid=(B // BT, N_pad // NT),
        in_specs=[
            pl.BlockSpec((BT, d), lambda i, j: (i, 0)),
            pl.BlockSpec((NT, d), lambda i, j: (j, 0)),
        ],
        out_specs=pl.BlockSpec((BT, NT), lambda i, j: (i, j)),
        out_shape=jax.ShapeDtypeStruct((B, N_pad), jnp.float32),
    )(k, cand_pad)

    _, context_idx = jax.lax.top_k(-dists, CONTEXT)
    context_k = candidate_embeddings[context_idx]
    context_y = candidate_y[context_idx]
    k_sq = jnp.sum(k * k, axis=-1, keepdims=True)
    cross = jnp.squeeze(
        jnp.matmul(k[:, None, :], jnp.swapaxes(context_k, -1, -2)), axis=-2)
    similarities = -k_sq + 2.0 * cross - jnp.sum(context_k * context_k, axis=-1)
    probs = jax.nn.softmax(similarities, axis=-1)
    context_y_emb = label_emb[context_y]
    diff = k[:, None, :] - context_k
    h = _layer_norm(diff, kp_ln_g, kp_ln_b)
    h = _silu(h @ kp_w1 + kp_b1) @ kp_w2 + kp_b2
    values = context_y_emb + diff + h
    context_x = jnp.squeeze(jnp.matmul(probs[:, None, :], values), axis=1)
    x = k + context_x
    y = _silu(x @ pr_w1 + pr_b1) @ pr_w2 + pr_b2
    return y
